# Initial kernel scaffold; baseline (speedup 1.0000x reference)
import jax
import jax.numpy as jnp
from jax.experimental import pallas as pl


def kernel(x, edge_index, W_l0, W_r0, b0, W_l1, W_r1, b1):
    raise NotImplementedError("write your pallas kernel here")


# speedup vs baseline: 7.1131x; 7.1131x over previous
import jax
import jax.numpy as jnp
from jax import lax
from jax.experimental import pallas as pl
from jax.experimental.pallas import tpu as pltpu
from jax.experimental.pallas import tpu_sc as plsc

N = 10000
D = 128
E = 320000
NSUB = 16
NW = 2 * NSUB
EPW = E // NW
C = 80
CH = EPW // C
NP = 10240
RPT = NP // NSUB
R = 1000
G = N // R


def _fill2d(ref, nrows, ncols, val):
    v16 = jnp.full((16,), val, jnp.float32)
    nc = ncols // 16

    def row(i, c):
        def col(k, c2):
            ref[i, pl.ds(k * 16, 16)] = v16
            return c2
        return lax.fori_loop(0, nc, col, c)

    lax.fori_loop(0, nrows, row, 0)


def _sc_body_count(ei_h, c_out, dstx, ones_v, zc_v, cnt_sh):
    ci = lax.axis_index("c")
    si = lax.axis_index("s")
    wid = ci * NSUB + si
    pltpu.sync_copy(ei_h.at[1, wid], dstx)
    _fill2d(ones_v, C, D, 1.0)
    _fill2d(zc_v, C, D, 0.0)
    base = si * RPT

    def zblk(j, c):
        pltpu.sync_copy(zc_v, cnt_sh.at[pl.ds(base + j * C, C)])
        return c

    lax.fori_loop(0, RPT // C, zblk, 0)
    plsc.subcore_barrier()

    def step(j, carry):
        pltpu.sync_copy(ones_v, cnt_sh.at[dstx.at[j]], add=True)
        return carry

    lax.fori_loop(0, CH, step, 0)
    plsc.subcore_barrier()
    pltpu.sync_copy(cnt_sh.at[pl.ds(base, RPT)], c_out.at[ci, pl.ds(base, RPT)])


def _sc_body(y_h, ei_h, s_out, srcx, dstx, rows, agg_sh, gsem):
    ci = lax.axis_index("c")
    si = lax.axis_index("s")
    wid = ci * NSUB + si
    pltpu.sync_copy(ei_h.at[0, wid], srcx)
    pltpu.sync_copy(ei_h.at[1, wid], dstx)
    _fill2d(rows, C, D, 0.0)
    base = si * RPT

    def zblk(j, c):
        pltpu.sync_copy(rows, agg_sh.at[pl.ds(base + j * C, C)])
        return c

    lax.fori_loop(0, RPT // C, zblk, 0)
    plsc.subcore_barrier()

    def step(j, carry):
        pltpu.async_copy(y_h.at[srcx.at[j]], rows, gsem).wait()
        pltpu.sync_copy(rows, agg_sh.at[dstx.at[j]], add=True)
        return carry

    lax.fori_loop(0, CH, step, 0)
    plsc.subcore_barrier()
    pltpu.sync_copy(agg_sh.at[pl.ds(base, RPT)], s_out.at[ci, pl.ds(base, RPT)])


def _make_sc_calls():
    mesh = plsc.VectorSubcoreMesh(core_axis_name="c", subcore_axis_name="s")
    count = pl.kernel(
        _sc_body_count,
        out_type=jax.ShapeDtypeStruct((2, NP, D), jnp.float32),
        mesh=mesh,
        scratch_types=[
            pltpu.VMEM((CH, C), jnp.int32),
            pltpu.VMEM((C, D), jnp.float32),
            pltpu.VMEM((C, D), jnp.float32),
            pltpu.VMEM_SHARED((NP, D), jnp.float32),
        ],
    )
    agg = pl.kernel(
        _sc_body,
        out_type=jax.ShapeDtypeStruct((2, NP, D), jnp.float32),
        mesh=mesh,
        scratch_types=[
            pltpu.VMEM((CH, C), jnp.int32),
            pltpu.VMEM((CH, C), jnp.int32),
            pltpu.VMEM((C, D), jnp.float32),
            pltpu.VMEM_SHARED((NP, D), jnp.float32),
            pltpu.SemaphoreType.DMA,
        ],
    )
    return count, agg


_SC_COUNT, _SC_AGG = _make_sc_calls()


def _mm2_body(x_ref, wl_ref, wr_ref, b_ref, y_ref, z_ref):
    xx = x_ref[...]
    y_ref[...] = jnp.dot(xx, wl_ref[...], preferred_element_type=jnp.float32)
    z_ref[...] = jnp.dot(xx, wr_ref[...], preferred_element_type=jnp.float32) + b_ref[...]


def _comb_mm_body(sp_ref, cp_ref, z_ref, wl_ref, wr_ref, b_ref, y_ref, z1_ref):
    s = sp_ref[0] + sp_ref[1]
    cnt = cp_ref[0, :, 0:1] + cp_ref[1, :, 0:1]
    h = jnp.maximum(s / jnp.maximum(cnt, 1.0) + z_ref[...], 0.0)
    y_ref[...] = jnp.dot(h, wl_ref[...], preferred_element_type=jnp.float32)
    z1_ref[...] = jnp.dot(h, wr_ref[...], preferred_element_type=jnp.float32) + b_ref[...]


def _final_body(sp_ref, cp_ref, z_ref, o_ref):
    s = sp_ref[0] + sp_ref[1]
    cnt = cp_ref[0, :, 0:1] + cp_ref[1, :, 0:1]
    o_ref[...] = s / jnp.maximum(cnt, 1.0) + z_ref[...]


_ROWS = pl.BlockSpec((R, D), lambda i: (i, 0))
_PARTS = pl.BlockSpec((2, R, D), lambda i: (0, i, 0))
_W = pl.BlockSpec((D, D), lambda i: (0, 0))
_B = pl.BlockSpec((1, D), lambda i: (0, 0))
_ROWS_OUT = jax.ShapeDtypeStruct((N, D), jnp.float32)


def _mm2(x, wl, wr, b):
    return pl.pallas_call(
        _mm2_body,
        grid=(G,),
        in_specs=[_ROWS, _W, _W, _B],
        out_specs=[_ROWS, _ROWS],
        out_shape=[_ROWS_OUT, _ROWS_OUT],
    )(x, wl, wr, b.reshape(1, D))


def _comb_mm(sp, cp, z, wl, wr, b):
    return pl.pallas_call(
        _comb_mm_body,
        grid=(G,),
        in_specs=[_PARTS, _PARTS, _ROWS, _W, _W, _B],
        out_specs=[_ROWS, _ROWS],
        out_shape=[_ROWS_OUT, _ROWS_OUT],
    )(sp, cp, z, wl, wr, b.reshape(1, D))


def _final(sp, cp, z):
    return pl.pallas_call(
        _final_body,
        grid=(G,),
        in_specs=[_PARTS, _PARTS, _ROWS],
        out_specs=_ROWS,
        out_shape=_ROWS_OUT,
    )(sp, cp, z)


def kernel(x, edge_index, W_l0, W_r0, b0, W_l1, W_r1, b1):
    ei_r = edge_index.reshape(2, NW, CH, C)

    c0 = _SC_COUNT(ei_r)
    y0, z0 = _mm2(x, W_l0, W_r0, b0)
    s0 = _SC_AGG(y0, ei_r)
    y1, z1 = _comb_mm(s0, c0, z0, W_l1, W_r1, b1)
    s1 = _SC_AGG(y1, ei_r)
    return _final(s1, c0, z1)
